# baseline (speedup 1.0000x reference)
import functools

import jax
import jax.numpy as jnp
from jax import lax
from jax.experimental import pallas as pl
from jax.experimental.pallas import tpu as pltpu
from jax.experimental.pallas import tpu_sc as plsc

_NC, _NS = 2, 16
_NW = _NC * _NS
_CH = 3


def kernel(inputs):
    B, H, W, C = inputs.shape
    HH, WW = H // 2, W // 2
    x6 = inputs.reshape(B, HH, 2, WW, 2, C)
    n_i = HH // _NW
    n_sub = n_i // _CH

    mesh = plsc.VectorSubcoreMesh(
        core_axis_name="c", subcore_axis_name="s",
        num_cores=_NC, num_subcores=_NS,
    )

    @functools.partial(
        pl.kernel,
        out_type=jax.ShapeDtypeStruct((B, HH, 2, WW, C), inputs.dtype),
        mesh=mesh,
        scratch_types=[
            pltpu.VMEM((_CH, WW, C), inputs.dtype),
            pltpu.VMEM((_CH, WW, C), inputs.dtype),
            pltpu.SemaphoreType.DMA,
            pltpu.SemaphoreType.DMA,
            pltpu.SemaphoreType.DMA,
            pltpu.SemaphoreType.DMA,
        ],
        compiler_params=pltpu.CompilerParams(use_tc_tiling_on_sc=False),
    )
    def seq_refresh(x_hbm, out_hbm, buf0, buf1, si0, si1, so0, so1):
        wid = lax.axis_index("s") * _NC + lax.axis_index("c")
        i_base = wid * n_i
        bufs = (buf0, buf1)
        sin = (si0, si1)
        sout = (so0, so1)
        units = [(b, q, s)
                 for b in range(B) for q in range(2) for s in range(n_sub)]

        def src(u):
            b, q, s = u
            return x_hbm.at[b, pl.ds(i_base + s * _CH, _CH), q, :, 1 - q, :]

        def dst(u):
            b, q, s = u
            return out_hbm.at[b, pl.ds(i_base + s * _CH, _CH), q, :, :]

        h_in = [None, None]
        h_out = [None, None]
        h_in[0] = pltpu.async_copy(src(units[0]), bufs[0], sin[0])
        for u in range(len(units)):
            k = u % 2
            k2 = (u + 1) % 2
            if u + 1 < len(units):
                if h_out[k2] is not None:
                    h_out[k2].wait()
                h_in[k2] = pltpu.async_copy(src(units[u + 1]), bufs[k2], sin[k2])
            h_in[k].wait()
            h_out[k] = pltpu.async_copy(bufs[k], dst(units[u]), sout[k])
        h_out[(len(units) - 2) % 2].wait()
        h_out[(len(units) - 1) % 2].wait()

    out5 = seq_refresh(x6)
    return out5.reshape(B, H * WW, C)


# speedup vs baseline: 1.6828x; 1.6828x over previous
import functools

import jax
import jax.numpy as jnp
from jax import lax
from jax.experimental import pallas as pl
from jax.experimental.pallas import tpu as pltpu
from jax.experimental.pallas import tpu_sc as plsc

_NC, _NS = 2, 16
_NW = _NC * _NS
_RPC = 128
_NB = 4
_D = 2


def kernel(inputs):
    B, H, W, C = inputs.shape
    WW = W // 2
    n_rows = B * H * WW
    table = inputs.reshape(B * H * W, C)
    rows_w = n_rows // _NW
    n_chunks = rows_w // _RPC

    o = jnp.arange(n_rows, dtype=jnp.int32)
    idx_all = (2 * o + 1 - ((o // WW) % 2)).reshape(_NW, n_chunks, _RPC)

    mesh = plsc.VectorSubcoreMesh(
        core_axis_name="c", subcore_axis_name="s",
        num_cores=_NC, num_subcores=_NS,
    )

    @functools.partial(
        pl.kernel,
        out_type=jax.ShapeDtypeStruct((n_rows, C), inputs.dtype),
        mesh=mesh,
        scratch_types=[
            pltpu.VMEM((n_chunks, _RPC), jnp.int32),
            *[pltpu.VMEM((_RPC, C), inputs.dtype) for _ in range(_NB)],
            *[pltpu.SemaphoreType.DMA for _ in range(2 * _NB)],
        ],
        compiler_params=pltpu.CompilerParams(use_tc_tiling_on_sc=False),
    )
    def seq_refresh(tab_hbm, idx_hbm, out_hbm, idx_v, *rest):
        bufs = rest[:_NB]
        sin = rest[_NB:2 * _NB]
        sout = rest[2 * _NB:]
        wid = lax.axis_index("s") * _NC + lax.axis_index("c")
        base = wid * rows_w

        pltpu.sync_copy(idx_hbm.at[wid], idx_v)

        g = [None] * n_chunks
        ocp = [None] * n_chunks
        for t in range(n_chunks):
            k = t % _NB
            if t >= _NB:
                ocp[t - _NB].wait()
            g[t] = pltpu.async_copy(tab_hbm.at[idx_v.at[t]], bufs[k], sin[k])
            td = t - _D
            if td >= 0:
                g[td].wait()
                ocp[td] = pltpu.async_copy(
                    bufs[td % _NB],
                    out_hbm.at[pl.ds(base + td * _RPC, _RPC), :],
                    sout[td % _NB])
        for t in range(n_chunks - _D, n_chunks):
            g[t].wait()
            ocp[t] = pltpu.async_copy(
                bufs[t % _NB],
                out_hbm.at[pl.ds(base + t * _RPC, _RPC), :],
                sout[t % _NB])
        for t in range(n_chunks - _NB, n_chunks):
            ocp[t].wait()

    out = seq_refresh(table, idx_all)
    return out.reshape(B, H * WW, C)
